# Initial kernel scaffold; baseline (speedup 1.0000x reference)
import jax
import jax.numpy as jnp
from jax.experimental import pallas as pl


def kernel(x, table, pe):
    raise NotImplementedError("write your pallas kernel here")


# speedup vs baseline: 2.8883x; 2.8883x over previous
import functools

import jax
import jax.numpy as jnp
from jax import lax
from jax.experimental import pallas as pl
from jax.experimental.pallas import tpu as pltpu
from jax.experimental.pallas import tpu_sc as plsc

S = 50
V = 39
D = 32
CHUNK = 512
NBUF = 2


@functools.lru_cache(maxsize=None)
def _build(n_tokens: int):
    mesh = plsc.VectorSubcoreMesh(core_axis_name="c", subcore_axis_name="s")
    nc, ns = mesh.num_cores, mesh.num_subcores
    nw = nc * ns
    assert n_tokens % (nw * CHUNK) == 0
    b_per_w = n_tokens // nw
    n_chunks = b_per_w // CHUNK

    def body(x_hbm, ft_hbm, out_hbm, idx_v, rows_v, in_sems, gat_sems, out_sems):
        wid = lax.axis_index("s") * nc + lax.axis_index("c")
        base = wid * b_per_w
        iota = lax.iota(jnp.int32, 16)

        def start_in(i):
            slot = lax.rem(i, NBUF)
            pltpu.async_copy(
                x_hbm.at[pl.ds(base + i * CHUNK, CHUNK)],
                idx_v.at[slot], in_sems.at[slot])

        def chunk_step(i, _):
            slot = lax.rem(i, NBUF)
            off = base + i * CHUNK
            pltpu.make_async_copy(
                x_hbm.at[pl.ds(off, CHUNK)], idx_v.at[slot],
                in_sems.at[slot]).wait()
            for g in range(CHUNK // 16):
                sl = pl.ds(g * 16, 16)
                pos = lax.rem(off + g * 16 + iota, S)
                idx_v[slot, sl] = idx_v[slot, sl] + pos * V
            @pl.when(i >= NBUF)
            def _():
                pltpu.make_async_copy(
                    rows_v.at[slot],
                    out_hbm.at[pl.ds(base + (i - NBUF) * CHUNK, CHUNK)],
                    out_sems.at[slot]).wait()
            pltpu.async_copy(ft_hbm.at[idx_v.at[slot]], rows_v.at[slot],
                             gat_sems.at[slot])
            pltpu.make_async_copy(ft_hbm.at[idx_v.at[slot]], rows_v.at[slot],
                                  gat_sems.at[slot]).wait()
            @pl.when(i + NBUF < n_chunks)
            def _():
                start_in(i + NBUF)
            pltpu.async_copy(rows_v.at[slot],
                             out_hbm.at[pl.ds(off, CHUNK)], out_sems.at[slot])
            return ()

        for b in range(min(NBUF, n_chunks)):
            start_in(b)
        lax.fori_loop(0, n_chunks, chunk_step, ())
        for b in range(min(NBUF, n_chunks)):
            i = n_chunks - min(NBUF, n_chunks) + b
            slot = lax.rem(jnp.int32(i), NBUF)
            pltpu.make_async_copy(
                rows_v.at[slot], out_hbm.at[pl.ds(base + i * CHUNK, CHUNK)],
                out_sems.at[slot]).wait()

    run = pl.kernel(
        body,
        out_type=jax.ShapeDtypeStruct((n_tokens, D), jnp.float32),
        mesh=mesh,
        scratch_types=[
            pltpu.VMEM((NBUF, CHUNK), jnp.int32),
            pltpu.VMEM((NBUF, CHUNK, D), jnp.float32),
            pltpu.SemaphoreType.DMA((NBUF,)),
            pltpu.SemaphoreType.DMA((NBUF,)),
            pltpu.SemaphoreType.DMA((NBUF,)),
        ],
        compiler_params=pltpu.CompilerParams(use_tc_tiling_on_sc=False),
    )
    return run


def kernel(x, table, pe):
    b, s = x.shape
    ft = (pe[0][:, None, :] + table[None, :, :]).reshape(S * V, D)
    out = _build(b * s)(x.reshape(-1), ft)
    return out.reshape(b, s, D)


# speedup vs baseline: 2.9019x; 1.0047x over previous
import functools

import jax
import jax.numpy as jnp
from jax import lax
from jax.experimental import pallas as pl
from jax.experimental.pallas import tpu as pltpu
from jax.experimental.pallas import tpu_sc as plsc

S = 50
V = 39
D = 32
CHUNK = 1024
NBUF = 2


@functools.lru_cache(maxsize=None)
def _build(n_tokens: int):
    mesh = plsc.VectorSubcoreMesh(core_axis_name="c", subcore_axis_name="s")
    nc, ns = mesh.num_cores, mesh.num_subcores
    nw = nc * ns
    assert n_tokens % (nw * CHUNK) == 0
    b_per_w = n_tokens // nw
    n_chunks = b_per_w // CHUNK

    def body(x_hbm, ft_hbm, out_hbm, idx_v, rows_v, in_sems, gat_sems, out_sems):
        wid = lax.axis_index("s") * nc + lax.axis_index("c")
        base = wid * b_per_w
        iota = lax.iota(jnp.int32, 16)

        def start_in(i):
            slot = lax.rem(i, NBUF)
            pltpu.async_copy(
                x_hbm.at[pl.ds(base + i * CHUNK, CHUNK)],
                idx_v.at[slot], in_sems.at[slot])

        def chunk_step(i, _):
            slot = lax.rem(i, NBUF)
            off = base + i * CHUNK
            pltpu.make_async_copy(
                x_hbm.at[pl.ds(off, CHUNK)], idx_v.at[slot],
                in_sems.at[slot]).wait()
            for g in range(CHUNK // 16):
                sl = pl.ds(g * 16, 16)
                pos = lax.rem(off + g * 16 + iota, S)
                idx_v[slot, sl] = idx_v[slot, sl] + pos * V
            @pl.when(i >= NBUF)
            def _():
                pltpu.make_async_copy(
                    rows_v.at[slot],
                    out_hbm.at[pl.ds(base + (i - NBUF) * CHUNK, CHUNK)],
                    out_sems.at[slot]).wait()
            pltpu.async_copy(ft_hbm.at[idx_v.at[slot]], rows_v.at[slot],
                             gat_sems.at[slot])
            pltpu.make_async_copy(ft_hbm.at[idx_v.at[slot]], rows_v.at[slot],
                                  gat_sems.at[slot]).wait()
            @pl.when(i + NBUF < n_chunks)
            def _():
                start_in(i + NBUF)
            pltpu.async_copy(rows_v.at[slot],
                             out_hbm.at[pl.ds(off, CHUNK)], out_sems.at[slot])
            return ()

        for b in range(min(NBUF, n_chunks)):
            start_in(b)
        lax.fori_loop(0, n_chunks, chunk_step, ())
        for b in range(min(NBUF, n_chunks)):
            i = n_chunks - min(NBUF, n_chunks) + b
            slot = lax.rem(jnp.int32(i), NBUF)
            pltpu.make_async_copy(
                rows_v.at[slot], out_hbm.at[pl.ds(base + i * CHUNK, CHUNK)],
                out_sems.at[slot]).wait()

    run = pl.kernel(
        body,
        out_type=jax.ShapeDtypeStruct((n_tokens, D), jnp.float32),
        mesh=mesh,
        scratch_types=[
            pltpu.VMEM((NBUF, CHUNK), jnp.int32),
            pltpu.VMEM((NBUF, CHUNK, D), jnp.float32),
            pltpu.SemaphoreType.DMA((NBUF,)),
            pltpu.SemaphoreType.DMA((NBUF,)),
            pltpu.SemaphoreType.DMA((NBUF,)),
        ],
        compiler_params=pltpu.CompilerParams(use_tc_tiling_on_sc=False),
    )
    return run


def kernel(x, table, pe):
    b, s = x.shape
    ft = (pe[0][:, None, :] + table[None, :, :]).reshape(S * V, D)
    out = _build(b * s)(x.reshape(-1), ft)
    return out.reshape(b, s, D)
